# baseline (speedup 1.0000x reference)
import functools

import jax
import jax.numpy as jnp
from jax import lax
from jax.experimental import pallas as pl
from jax.experimental.pallas import tpu as pltpu
from jax.experimental.pallas import tpu_sc as plsc

BATCH = 4096
HIST = 50
EMBED_DIM = 64
LANES = 16
NUM_WORKERS = 32
B_PER_W = BATCH // NUM_WORKERS
GROUPS = B_PER_W // LANES
PAD_DIM = 2 * EMBED_DIM


def _worker_id():
    return lax.axis_index("s") * 2 + lax.axis_index("c")


def _kernel_a(user_ids_hbm, item_seq_t_hbm, last_pos_hbm,
              user_table2_hbm, global_emb_hbm,
              partial_hbm, iids_hbm,
              uid_v, lp_v, seq_v, iid_v, urows_v, part_v, g_v,
              sem_u):
    base = _worker_id() * B_PER_W

    pltpu.sync_copy(user_ids_hbm.at[pl.ds(base, B_PER_W)], uid_v)
    pltpu.sync_copy(last_pos_hbm.at[pl.ds(base, B_PER_W)], lp_v)
    pltpu.sync_copy(item_seq_t_hbm.at[:, pl.ds(base, B_PER_W)], seq_v)
    pltpu.sync_copy(global_emb_hbm, g_v)

    lane_ids = lax.iota(jnp.int32, LANES)

    def fire(g, carry):
        gbase = g * LANES
        uc = uid_v[pl.ds(gbase, LANES)]
        lpc = lp_v[pl.ds(gbase, LANES)]
        vals = jnp.zeros((LANES,), jnp.int32)
        for i in range(LANES):
            pltpu.async_copy(user_table2_hbm.at[uc[i] // 2],
                             urows_v.at[gbase + i], sem_u)
            iid_s = seq_v[lpc[i], pl.ds(gbase, LANES)][i]
            vals = jnp.where(lane_ids == i, jnp.full((LANES,), iid_s), vals)
        iid_v[pl.ds(gbase, LANES)] = vals
        return carry

    lax.fori_loop(0, GROUPS, fire, 0)

    pltpu.sync_copy(iid_v, iids_hbm.at[pl.ds(base, B_PER_W)])

    pltpu.make_async_copy(
        user_table2_hbm.at[pl.ds(0, B_PER_W), :], urows_v, sem_u).wait()

    g_chunks = [g_v[pl.ds(c * LANES, LANES)] for c in range(EMBED_DIM // LANES)]
    one = jnp.float32(1.0)
    zero = jnp.float32(0.0)

    def body(g, carry):
        gbase = g * LANES
        uc = uid_v[pl.ds(gbase, LANES)]
        su = jnp.where(uc != 0, one, zero)
        for i in range(LANES):
            e = gbase + i
            s_u = jnp.full((LANES,), su[i], dtype=jnp.float32)
            off = (uc[i] % 2) * EMBED_DIM
            for c in range(EMBED_DIM // LANES):
                part_v[e, pl.ds(c * LANES, LANES)] = (
                    urows_v[e, pl.ds(off + c * LANES, LANES)] * s_u
                    + g_chunks[c])
        return carry

    lax.fori_loop(0, GROUPS, body, 0)

    pltpu.sync_copy(part_v, partial_hbm.at[pl.ds(base, B_PER_W), :])


def _kernel_b(iids_hbm, item_table2_hbm, partial_hbm,
              out_hbm,
              iid_v, irows_v, part_v, out_v,
              sem_i):
    base = _worker_id() * B_PER_W

    pltpu.sync_copy(iids_hbm.at[pl.ds(base, B_PER_W)], iid_v)

    def fire(g, carry):
        ic = iid_v[pl.ds(g * LANES, LANES)]
        for i in range(LANES):
            pltpu.async_copy(item_table2_hbm.at[ic[i] // 2],
                             irows_v.at[g * LANES + i], sem_i)
        return carry

    lax.fori_loop(0, GROUPS, fire, 0)

    pltpu.sync_copy(partial_hbm.at[pl.ds(base, B_PER_W), :], part_v)
    pltpu.make_async_copy(
        item_table2_hbm.at[pl.ds(0, B_PER_W), :], irows_v, sem_i).wait()

    one = jnp.float32(1.0)
    zero = jnp.float32(0.0)

    def body(g, carry):
        gbase = g * LANES
        ic = iid_v[pl.ds(gbase, LANES)]
        si = jnp.where(ic != 0, one, zero)
        for i in range(LANES):
            e = gbase + i
            s_i = jnp.full((LANES,), si[i], dtype=jnp.float32)
            off = (ic[i] % 2) * EMBED_DIM
            for c in range(EMBED_DIM // LANES):
                out_v[e, pl.ds(c * LANES, LANES)] = (
                    part_v[e, pl.ds(c * LANES, LANES)]
                    + irows_v[e, pl.ds(off + c * LANES, LANES)] * s_i)
        return carry

    lax.fori_loop(0, GROUPS, body, 0)

    pltpu.sync_copy(out_v, out_hbm.at[pl.ds(base, B_PER_W), :])


@jax.jit
def _run(user_ids, item_seq, last_pos, user_table, item_table, global_user_emb):
    mesh = plsc.VectorSubcoreMesh(core_axis_name="c", subcore_axis_name="s")

    fa = functools.partial(
        pl.kernel,
        mesh=mesh,
        out_type=(
            jax.ShapeDtypeStruct((BATCH, PAD_DIM), jnp.float32),
            jax.ShapeDtypeStruct((BATCH,), jnp.int32),
        ),
        scratch_types=[
            pltpu.VMEM((B_PER_W,), jnp.int32),
            pltpu.VMEM((B_PER_W,), jnp.int32),
            pltpu.VMEM((HIST, B_PER_W), jnp.int32),
            pltpu.VMEM((B_PER_W,), jnp.int32),
            pltpu.VMEM((B_PER_W, PAD_DIM), jnp.float32),
            pltpu.VMEM((B_PER_W, PAD_DIM), jnp.float32),
            pltpu.VMEM((EMBED_DIM,), jnp.float32),
            pltpu.SemaphoreType.DMA,
        ],
    )(_kernel_a)

    fb = functools.partial(
        pl.kernel,
        mesh=mesh,
        out_type=jax.ShapeDtypeStruct((BATCH, EMBED_DIM), jnp.float32),
        scratch_types=[
            pltpu.VMEM((B_PER_W,), jnp.int32),
            pltpu.VMEM((B_PER_W, PAD_DIM), jnp.float32),
            pltpu.VMEM((B_PER_W, PAD_DIM), jnp.float32),
            pltpu.VMEM((B_PER_W, EMBED_DIM), jnp.float32),
            pltpu.SemaphoreType.DMA,
        ],
    )(_kernel_b)

    user_table2 = user_table.reshape(-1, PAD_DIM)
    item_table2 = item_table.reshape(-1, PAD_DIM)
    partial, iids = fa(user_ids.astype(jnp.int32),
                       item_seq.astype(jnp.int32).T,
                       last_pos.astype(jnp.int32),
                       user_table2, global_user_emb)
    return fb(iids, item_table2, partial)


def kernel(user_ids, item_seq, last_pos, user_table, item_table, global_user_emb):
    return _run(user_ids, item_seq, last_pos, user_table, item_table,
                global_user_emb)


# speedup vs baseline: 1.4513x; 1.4513x over previous
import functools

import jax
import jax.numpy as jnp
from jax import lax
from jax.experimental import pallas as pl
from jax.experimental.pallas import tpu as pltpu
from jax.experimental.pallas import tpu_sc as plsc

BATCH = 4096
HIST = 50
EMBED_DIM = 64
LANES = 16
NUM_WORKERS = 32
B_PER_W = BATCH // NUM_WORKERS
GROUPS = B_PER_W // LANES
PAD_DIM = 2 * EMBED_DIM


def _worker_id():
    return lax.axis_index("s") * 2 + lax.axis_index("c")


def _kernel_a(user_ids_hbm, item_seq_t_hbm, last_pos_hbm,
              user_table_hbm, global_emb_hbm,
              partial_hbm, iids_hbm,
              uid_v, lp_v, seq_v, iid_v, urows_v, part_v, g_v,
              sem_u):
    base = _worker_id() * B_PER_W

    pltpu.sync_copy(user_ids_hbm.at[pl.ds(base, B_PER_W)], uid_v)
    pltpu.sync_copy(last_pos_hbm.at[pl.ds(base, B_PER_W)], lp_v)
    pltpu.sync_copy(item_seq_t_hbm.at[:, pl.ds(base, B_PER_W)], seq_v)
    pltpu.sync_copy(global_emb_hbm, g_v)

    lane_ids = lax.iota(jnp.int32, LANES)

    def fire(g, carry):
        gbase = g * LANES
        uc = uid_v[pl.ds(gbase, LANES)]
        lpc = lp_v[pl.ds(gbase, LANES)]
        vals = jnp.zeros((LANES,), jnp.int32)
        for i in range(LANES):
            pltpu.async_copy(user_table_hbm.at[uc[i]],
                             urows_v.at[gbase + i], sem_u)
            iid_s = seq_v[lpc[i], pl.ds(gbase, LANES)][i]
            vals = jnp.where(lane_ids == i, jnp.full((LANES,), iid_s), vals)
        iid_v[pl.ds(gbase, LANES)] = vals
        return carry

    lax.fori_loop(0, GROUPS, fire, 0)

    pltpu.sync_copy(iid_v, iids_hbm.at[pl.ds(base, B_PER_W)])

    pltpu.make_async_copy(
        user_table_hbm.at[pl.ds(0, B_PER_W), :], urows_v, sem_u).wait()

    g_chunks = [g_v[pl.ds(c * LANES, LANES)] for c in range(EMBED_DIM // LANES)]
    one = jnp.float32(1.0)
    zero = jnp.float32(0.0)

    def body(g, carry):
        su = jnp.where(uid_v[pl.ds(g * LANES, LANES)] != 0, one, zero)
        for i in range(LANES):
            e = g * LANES + i
            s_u = jnp.full((LANES,), su[i], dtype=jnp.float32)
            for c in range(EMBED_DIM // LANES):
                part_v[e, pl.ds(c * LANES, LANES)] = (
                    urows_v[e, pl.ds(c * LANES, LANES)] * s_u + g_chunks[c])
        return carry

    lax.fori_loop(0, GROUPS, body, 0)

    pltpu.sync_copy(part_v, partial_hbm.at[pl.ds(base, B_PER_W), :])


def _kernel_b(iids_hbm, item_table_hbm, partial_hbm,
              out_hbm,
              iid_v, irows_v, part_v, out_v,
              sem_i):
    base = _worker_id() * B_PER_W

    pltpu.sync_copy(iids_hbm.at[pl.ds(base, B_PER_W)], iid_v)

    def fire(g, carry):
        ic = iid_v[pl.ds(g * LANES, LANES)]
        for i in range(LANES):
            pltpu.async_copy(item_table_hbm.at[ic[i]],
                             irows_v.at[g * LANES + i], sem_i)
        return carry

    lax.fori_loop(0, GROUPS, fire, 0)

    pltpu.sync_copy(partial_hbm.at[pl.ds(base, B_PER_W), :], part_v)
    pltpu.make_async_copy(
        item_table_hbm.at[pl.ds(0, B_PER_W), :], irows_v, sem_i).wait()

    one = jnp.float32(1.0)
    zero = jnp.float32(0.0)

    def body(g, carry):
        si = jnp.where(iid_v[pl.ds(g * LANES, LANES)] != 0, one, zero)
        for i in range(LANES):
            e = g * LANES + i
            s_i = jnp.full((LANES,), si[i], dtype=jnp.float32)
            for c in range(EMBED_DIM // LANES):
                out_v[e, pl.ds(c * LANES, LANES)] = (
                    part_v[e, pl.ds(c * LANES, LANES)]
                    + irows_v[e, pl.ds(c * LANES, LANES)] * s_i)
        return carry

    lax.fori_loop(0, GROUPS, body, 0)

    pltpu.sync_copy(out_v, out_hbm.at[pl.ds(base, B_PER_W), :])


@jax.jit
def _run(user_ids, item_seq, last_pos, user_table, item_table, global_user_emb):
    mesh = plsc.VectorSubcoreMesh(core_axis_name="c", subcore_axis_name="s")

    fa = functools.partial(
        pl.kernel,
        mesh=mesh,
        out_type=(
            jax.ShapeDtypeStruct((BATCH, PAD_DIM), jnp.float32),
            jax.ShapeDtypeStruct((BATCH,), jnp.int32),
        ),
        scratch_types=[
            pltpu.VMEM((B_PER_W,), jnp.int32),
            pltpu.VMEM((B_PER_W,), jnp.int32),
            pltpu.VMEM((HIST, B_PER_W), jnp.int32),
            pltpu.VMEM((B_PER_W,), jnp.int32),
            pltpu.VMEM((B_PER_W, EMBED_DIM), jnp.float32),
            pltpu.VMEM((B_PER_W, PAD_DIM), jnp.float32),
            pltpu.VMEM((EMBED_DIM,), jnp.float32),
            pltpu.SemaphoreType.DMA,
        ],
    )(_kernel_a)

    fb = functools.partial(
        pl.kernel,
        mesh=mesh,
        out_type=jax.ShapeDtypeStruct((BATCH, EMBED_DIM), jnp.float32),
        scratch_types=[
            pltpu.VMEM((B_PER_W,), jnp.int32),
            pltpu.VMEM((B_PER_W, EMBED_DIM), jnp.float32),
            pltpu.VMEM((B_PER_W, PAD_DIM), jnp.float32),
            pltpu.VMEM((B_PER_W, EMBED_DIM), jnp.float32),
            pltpu.SemaphoreType.DMA,
        ],
    )(_kernel_b)

    partial, iids = fa(user_ids.astype(jnp.int32),
                       item_seq.astype(jnp.int32).T,
                       last_pos.astype(jnp.int32),
                       user_table, global_user_emb)
    return fb(iids, item_table, partial)


def kernel(user_ids, item_seq, last_pos, user_table, item_table, global_user_emb):
    return _run(user_ids, item_seq, last_pos, user_table, item_table,
                global_user_emb)
